# Initial kernel scaffold; baseline (speedup 1.0000x reference)
import jax
import jax.numpy as jnp
from jax.experimental import pallas as pl


def kernel(x, edge_index, batch, W0, b0, bn0_g, bn0_b, lin0_W, lin0_b, c1_W, c1_b, bn1_g, bn1_b, lin1_W, lin1_b, c2_W, c2_b, bn2_g, bn2_b, lin2_W, lin2_b):
    raise NotImplementedError("write your pallas kernel here")


# speedup vs baseline: 21.1717x; 21.1717x over previous
import functools

import jax
import jax.numpy as jnp
from jax import lax
from jax.experimental import pallas as pl
from jax.experimental.pallas import tpu as pltpu
from jax.experimental.pallas import tpu_sc as plsc

_N = 100000
_F = 128
_E = 3200000
_G = 8
_T = 4

_NC = 2
_NS = 16
_NW = _NC * _NS
_KCH = 8
_EPT_STEP = _KCH * 128
_NITER = -(-_E // (_NW * _EPT_STEP))
_E_PAD = _NITER * _NW * _EPT_STEP
_ROWS_PER_TILE = _NITER * _KCH
_PER_T = 6251
_N_ACC = _NS * _PER_T
_ZROWS = 256
_CHUNKS = [(o, min(_ZROWS, _PER_T - o)) for o in range(0, _PER_T, _ZROWS)]

_R = 2000
_INV_SQRT2 = 0.7071067811865476


def _gelu(v):
    return 0.5 * v * (1.0 + lax.erf(v * _INV_SQRT2))


def _sc_agg_call(h, src2, dst2):
    mesh = plsc.VectorSubcoreMesh(core_axis_name="c", subcore_axis_name="s")

    @functools.partial(
        pl.kernel,
        out_type=jax.ShapeDtypeStruct((_NC, _N_ACC, 16), jnp.float32),
        mesh=mesh,
        scratch_types=[
            pltpu.VMEM((_KCH, 128), jnp.int32),
            pltpu.VMEM((_KCH, 128), jnp.int32),
            pltpu.VMEM((_KCH, 128, 16), jnp.float32),
            pltpu.VMEM((_ZROWS, 16), jnp.float32),
            pltpu.VMEM_SHARED((_N_ACC, 16), jnp.float32),
            pltpu.SemaphoreType.DMA,
        ],
        compiler_params=pltpu.CompilerParams(use_tc_tiling_on_sc=False),
    )
    def body(h_hbm, src_hbm, dst_hbm, out_hbm, src_v, dst_v, rows_v, zbuf, acc, sem):
        cid = lax.axis_index("c")
        sid = lax.axis_index("s")
        wid = sid * _NC + cid

        def zrow(i, carry):
            zbuf[i, :] = jnp.zeros((16,), jnp.float32)
            return carry

        lax.fori_loop(0, _ZROWS, zrow, 0)
        for off, sz in _CHUNKS:
            pltpu.sync_copy(zbuf.at[pl.ds(0, sz)],
                            acc.at[pl.ds(sid * _PER_T + off, sz)])
        plsc.subcore_barrier()

        def step(it, carry):
            row0 = wid * _ROWS_PER_TILE + it * _KCH
            pltpu.sync_copy(src_hbm.at[pl.ds(row0, _KCH)], src_v)
            pltpu.sync_copy(dst_hbm.at[pl.ds(row0, _KCH)], dst_v)
            copies = [
                pltpu.async_copy(h_hbm.at[src_v.at[j]], rows_v.at[j], sem)
                for j in range(_KCH)
            ]
            for c in copies:
                c.wait()
            for j in range(_KCH):
                pltpu.sync_copy(rows_v.at[j], acc.at[dst_v.at[j]], add=True)
            return carry

        lax.fori_loop(0, _NITER, step, 0)
        plsc.subcore_barrier()
        for off, sz in _CHUNKS:
            sl = pl.ds(sid * _PER_T + off, sz)
            pltpu.sync_copy(acc.at[sl], out_hbm.at[cid].at[sl])

    return body(h, src2, dst2)


def _p1_first_body(x_ref, w_ref, y_ref, st_ref):
    i = pl.program_id(0)
    y = jnp.dot(x_ref[...], w_ref[...], preferred_element_type=jnp.float32)
    y_ref[...] = y
    s = jnp.sum(y, axis=0, keepdims=True)
    sq = jnp.sum(y * y, axis=0, keepdims=True)
    upd = jnp.concatenate([s, sq, jnp.zeros((6, y.shape[1]), jnp.float32)], 0)

    @pl.when(i == 0)
    def _():
        st_ref[...] = jnp.zeros_like(st_ref)

    st_ref[...] += upd


def _p1_gin_body(h_ref, agga_ref, aggb_ref, w_ref, y_ref, st_ref):
    i = pl.program_id(0)
    hin = h_ref[...] + agga_ref[0] + aggb_ref[0]
    y = jnp.dot(hin, w_ref[...], preferred_element_type=jnp.float32)
    y_ref[...] = y
    s = jnp.sum(y, axis=0, keepdims=True)
    sq = jnp.sum(y * y, axis=0, keepdims=True)
    upd = jnp.concatenate([s, sq, jnp.zeros((6, y.shape[1]), jnp.float32)], 0)

    @pl.when(i == 0)
    def _():
        st_ref[...] = jnp.zeros_like(st_ref)

    st_ref[...] += upd


def _p2_body(gelu_on, has_zp, *refs):
    if has_zp:
        (y_ref, st_ref, g_ref, b_ref, lw_ref, lb_ref, bat_ref, zp_ref,
         h_ref, z_ref, seg_ref) = refs
    else:
        (y_ref, st_ref, g_ref, b_ref, lw_ref, lb_ref, bat_ref,
         h_ref, z_ref, seg_ref) = refs
        zp_ref = None
    i = pl.program_id(0)
    st = st_ref[...]
    inv_n = 1.0 / _N
    m = st[0:1, :] * inv_n
    v = st[1:2, :] * inv_n - m * m
    scale = g_ref[...] * lax.rsqrt(v + 1e-5)
    h = (y_ref[...] - m) * scale + b_ref[...]
    if gelu_on:
        h = _gelu(h)
    h_ref[...] = h
    z = jnp.dot(h, lw_ref[...], preferred_element_type=jnp.float32) + lb_ref[...]
    if gelu_on:
        z = _gelu(z)
    if zp_ref is not None:
        z_ref[...] = z + zp_ref[...]
    else:
        z_ref[...] = z
    bat = bat_ref[...]
    rows = [
        jnp.max(jnp.where(bat == g, z, -jnp.inf), axis=0, keepdims=True)
        for g in range(_G)
    ]
    cur = jnp.concatenate(rows, 0)

    @pl.when(i == 0)
    def _():
        seg_ref[...] = jnp.full_like(seg_ref, -jnp.inf)

    seg_ref[...] = jnp.maximum(seg_ref[...], cur)


_SEQ = pltpu.CompilerParams(dimension_semantics=("arbitrary",))


def _pass1_first(x, w):
    grid = _N // _R
    return pl.pallas_call(
        _p1_first_body,
        grid=(grid,),
        in_specs=[
            pl.BlockSpec((_R, _F), lambda i: (i, 0)),
            pl.BlockSpec((_F, 16), lambda i: (0, 0)),
        ],
        out_specs=[
            pl.BlockSpec((_R, 16), lambda i: (i, 0)),
            pl.BlockSpec((8, 16), lambda i: (0, 0)),
        ],
        out_shape=[
            jax.ShapeDtypeStruct((_N, 16), jnp.float32),
            jax.ShapeDtypeStruct((8, 16), jnp.float32),
        ],
        compiler_params=_SEQ,
    )(x, w)


def _pass1_gin(h, agg, w, dout):
    grid = _N // _R
    return pl.pallas_call(
        _p1_gin_body,
        grid=(grid,),
        in_specs=[
            pl.BlockSpec((_R, 16), lambda i: (i, 0)),
            pl.BlockSpec((1, _R, 16), lambda i: (0, i, 0)),
            pl.BlockSpec((1, _R, 16), lambda i: (1, i, 0)),
            pl.BlockSpec((16, dout), lambda i: (0, 0)),
        ],
        out_specs=[
            pl.BlockSpec((_R, dout), lambda i: (i, 0)),
            pl.BlockSpec((8, dout), lambda i: (0, 0)),
        ],
        out_shape=[
            jax.ShapeDtypeStruct((_N, dout), jnp.float32),
            jax.ShapeDtypeStruct((8, dout), jnp.float32),
        ],
        compiler_params=_SEQ,
    )(h, agg, agg, w)


def _pass2(y, st, g, b, lw, lb, bat2, zp, dout, gelu_on):
    grid = _N // _R
    in_specs = [
        pl.BlockSpec((_R, dout), lambda i: (i, 0)),
        pl.BlockSpec((8, dout), lambda i: (0, 0)),
        pl.BlockSpec((1, dout), lambda i: (0, 0)),
        pl.BlockSpec((1, dout), lambda i: (0, 0)),
        pl.BlockSpec((dout, _T), lambda i: (0, 0)),
        pl.BlockSpec((1, _T), lambda i: (0, 0)),
        pl.BlockSpec((_R, 1), lambda i: (i, 0)),
    ]
    args = [y, st, g.reshape(1, dout), b.reshape(1, dout), lw,
            lb.reshape(1, _T), bat2]
    if zp is not None:
        in_specs.append(pl.BlockSpec((_R, _T), lambda i: (i, 0)))
        args.append(zp)
    return pl.pallas_call(
        functools.partial(_p2_body, gelu_on, zp is not None),
        grid=(grid,),
        in_specs=in_specs,
        out_specs=[
            pl.BlockSpec((_R, dout), lambda i: (i, 0)),
            pl.BlockSpec((_R, _T), lambda i: (i, 0)),
            pl.BlockSpec((_G, _T), lambda i: (0, 0)),
        ],
        out_shape=[
            jax.ShapeDtypeStruct((_N, dout), jnp.float32),
            jax.ShapeDtypeStruct((_N, _T), jnp.float32),
            jax.ShapeDtypeStruct((_G, _T), jnp.float32),
        ],
        compiler_params=_SEQ,
    )(*args)


def kernel(x, edge_index, batch, W0, b0, bn0_g, bn0_b, lin0_W, lin0_b,
           c1_W, c1_b, bn1_g, bn1_b, lin1_W, lin1_b,
           c2_W, c2_b, bn2_g, bn2_b, lin2_W, lin2_b):
    src = edge_index[0]
    dst = edge_index[1]
    pad = _E_PAD - _E
    src2 = jnp.concatenate([src, jnp.zeros((pad,), jnp.int32)]).reshape(-1, 128)
    dst2 = jnp.concatenate([dst, jnp.full((pad,), _N, jnp.int32)]).reshape(-1, 128)
    bat2 = batch.reshape(_N, 1)

    y0, st0 = _pass1_first(x, W0)
    h0, z0, seg0 = _pass2(y0, st0, bn0_g, bn0_b, lin0_W, lin0_b, bat2,
                          None, 16, True)

    agg1 = _sc_agg_call(h0, src2, dst2)
    y1, st1 = _pass1_gin(h0, agg1, c1_W, 16)
    h1, z1, seg1 = _pass2(y1, st1, bn1_g, bn1_b, lin1_W, lin1_b, bat2,
                          z0, 16, False)

    agg2 = _sc_agg_call(h1, src2, dst2)
    y2, st2 = _pass1_gin(h1, agg2, c2_W, 8)
    h2, z2, seg2 = _pass2(y2, st2, bn2_g, bn2_b, lin2_W, lin2_b, bat2,
                          z1, 8, False)

    out = seg0 + seg1 + seg2
    return (out, z2, h2)


# speedup vs baseline: 22.6782x; 1.0712x over previous
import functools

import jax
import jax.numpy as jnp
from jax import lax
from jax.experimental import pallas as pl
from jax.experimental.pallas import tpu as pltpu
from jax.experimental.pallas import tpu_sc as plsc

_N = 100000
_F = 128
_E = 3200000
_G = 8
_T = 4

_NC = 2
_NS = 16
_NW = _NC * _NS
_KCH = 8
_EROWS = _E // 128
_RPT = _EROWS // _NW
_RREM = _EROWS % _NW
_PER_T = 6250
_N_ACC = _NS * _PER_T
_ZROWS = 250
_CHUNKS = [(o, _ZROWS) for o in range(0, _PER_T, _ZROWS)]

_R = 2000
_INV_SQRT2 = 0.7071067811865476


def _gelu(v):
    return 0.5 * v * (1.0 + lax.erf(v * _INV_SQRT2))


def _sc_agg_call(h, edges3):
    mesh = plsc.VectorSubcoreMesh(core_axis_name="c", subcore_axis_name="s")

    @functools.partial(
        pl.kernel,
        out_type=jax.ShapeDtypeStruct((_NC, _N_ACC, 16), jnp.float32),
        mesh=mesh,
        scratch_types=[
            pltpu.VMEM((_KCH, 128), jnp.int32),
            pltpu.VMEM((_KCH, 128), jnp.int32),
            pltpu.VMEM((_KCH, 128, 16), jnp.float32),
            pltpu.VMEM((_ZROWS, 16), jnp.float32),
            pltpu.VMEM_SHARED((_N_ACC, 16), jnp.float32),
            pltpu.SemaphoreType.DMA,
        ],
        compiler_params=pltpu.CompilerParams(use_tc_tiling_on_sc=False),
    )
    def body(h_hbm, e_hbm, out_hbm, src_v, dst_v, rows_v, zbuf, acc, sem):
        cid = lax.axis_index("c")
        sid = lax.axis_index("s")
        wid = sid * _NC + cid

        def zrow(i, carry):
            zbuf[i, :] = jnp.zeros((16,), jnp.float32)
            return carry

        lax.fori_loop(0, _ZROWS, zrow, 0)
        for off, sz in _CHUNKS:
            pltpu.sync_copy(zbuf.at[pl.ds(0, sz)],
                            acc.at[pl.ds(sid * _PER_T + off, sz)])
        plsc.subcore_barrier()

        start = wid * _RPT + jnp.minimum(wid, _RREM)
        count = _RPT + (wid < _RREM).astype(jnp.int32)
        nfull = count // _KCH

        def gather_scatter(row0, k):
            pltpu.sync_copy(e_hbm.at[0, pl.ds(row0, k)], src_v.at[pl.ds(0, k)])
            pltpu.sync_copy(e_hbm.at[1, pl.ds(row0, k)], dst_v.at[pl.ds(0, k)])
            copies = [
                pltpu.async_copy(h_hbm.at[src_v.at[j]], rows_v.at[j], sem)
                for j in range(k)
            ]
            for c in copies:
                c.wait()
            for j in range(k):
                pltpu.sync_copy(rows_v.at[j], acc.at[dst_v.at[j]], add=True)

        def step(it, carry):
            gather_scatter(start + it * _KCH, _KCH)
            return carry

        lax.fori_loop(0, nfull, step, 0)

        def tail(it, carry):
            gather_scatter(start + nfull * _KCH + it, 1)
            return carry

        lax.fori_loop(0, count - nfull * _KCH, tail, 0)
        plsc.subcore_barrier()
        for off, sz in _CHUNKS:
            sl = pl.ds(sid * _PER_T + off, sz)
            pltpu.sync_copy(acc.at[sl], out_hbm.at[cid].at[sl])

    return body(h, edges3)


def _p1_first_body(x_ref, w_ref, y_ref, st_ref):
    i = pl.program_id(0)
    y = jnp.dot(x_ref[...], w_ref[...], preferred_element_type=jnp.float32)
    y_ref[...] = y
    s = jnp.sum(y, axis=0, keepdims=True)
    sq = jnp.sum(y * y, axis=0, keepdims=True)
    upd = jnp.concatenate([s, sq, jnp.zeros((6, y.shape[1]), jnp.float32)], 0)

    @pl.when(i == 0)
    def _():
        st_ref[...] = jnp.zeros_like(st_ref)

    st_ref[...] += upd


def _p1_gin_body(h_ref, agga_ref, aggb_ref, w_ref, y_ref, st_ref):
    i = pl.program_id(0)
    hin = h_ref[...] + agga_ref[0] + aggb_ref[0]
    y = jnp.dot(hin, w_ref[...], preferred_element_type=jnp.float32)
    y_ref[...] = y
    s = jnp.sum(y, axis=0, keepdims=True)
    sq = jnp.sum(y * y, axis=0, keepdims=True)
    upd = jnp.concatenate([s, sq, jnp.zeros((6, y.shape[1]), jnp.float32)], 0)

    @pl.when(i == 0)
    def _():
        st_ref[...] = jnp.zeros_like(st_ref)

    st_ref[...] += upd


def _p2_body(gelu_on, has_zp, *refs):
    if has_zp:
        (y_ref, st_ref, g_ref, b_ref, lw_ref, lb_ref, bat_ref, zp_ref,
         h_ref, z_ref, seg_ref) = refs
    else:
        (y_ref, st_ref, g_ref, b_ref, lw_ref, lb_ref, bat_ref,
         h_ref, z_ref, seg_ref) = refs
        zp_ref = None
    i = pl.program_id(0)
    st = st_ref[...]
    inv_n = 1.0 / _N
    m = st[0:1, :] * inv_n
    v = st[1:2, :] * inv_n - m * m
    scale = g_ref[...] * lax.rsqrt(v + 1e-5)
    h = (y_ref[...] - m) * scale + b_ref[...]
    if gelu_on:
        h = _gelu(h)
    h_ref[...] = h
    z = jnp.dot(h, lw_ref[...], preferred_element_type=jnp.float32) + lb_ref[...]
    if gelu_on:
        z = _gelu(z)
    if zp_ref is not None:
        z_ref[...] = z + zp_ref[...]
    else:
        z_ref[...] = z
    bat = bat_ref[...]
    rows = [
        jnp.max(jnp.where(bat == g, z, -jnp.inf), axis=0, keepdims=True)
        for g in range(_G)
    ]
    cur = jnp.concatenate(rows, 0)

    @pl.when(i == 0)
    def _():
        seg_ref[...] = jnp.full_like(seg_ref, -jnp.inf)

    seg_ref[...] = jnp.maximum(seg_ref[...], cur)


_SEQ = pltpu.CompilerParams(dimension_semantics=("arbitrary",))


def _pass1_first(x, w):
    grid = _N // _R
    return pl.pallas_call(
        _p1_first_body,
        grid=(grid,),
        in_specs=[
            pl.BlockSpec((_R, _F), lambda i: (i, 0)),
            pl.BlockSpec((_F, 16), lambda i: (0, 0)),
        ],
        out_specs=[
            pl.BlockSpec((_R, 16), lambda i: (i, 0)),
            pl.BlockSpec((8, 16), lambda i: (0, 0)),
        ],
        out_shape=[
            jax.ShapeDtypeStruct((_N, 16), jnp.float32),
            jax.ShapeDtypeStruct((8, 16), jnp.float32),
        ],
        compiler_params=_SEQ,
    )(x, w)


def _pass1_gin(h, agg, w, dout):
    grid = _N // _R
    return pl.pallas_call(
        _p1_gin_body,
        grid=(grid,),
        in_specs=[
            pl.BlockSpec((_R, 16), lambda i: (i, 0)),
            pl.BlockSpec((1, _R, 16), lambda i: (0, i, 0)),
            pl.BlockSpec((1, _R, 16), lambda i: (1, i, 0)),
            pl.BlockSpec((16, dout), lambda i: (0, 0)),
        ],
        out_specs=[
            pl.BlockSpec((_R, dout), lambda i: (i, 0)),
            pl.BlockSpec((8, dout), lambda i: (0, 0)),
        ],
        out_shape=[
            jax.ShapeDtypeStruct((_N, dout), jnp.float32),
            jax.ShapeDtypeStruct((8, dout), jnp.float32),
        ],
        compiler_params=_SEQ,
    )(h, agg, agg, w)


def _pass2(y, st, g, b, lw, lb, bat2, zp, dout, gelu_on):
    grid = _N // _R
    in_specs = [
        pl.BlockSpec((_R, dout), lambda i: (i, 0)),
        pl.BlockSpec((8, dout), lambda i: (0, 0)),
        pl.BlockSpec((1, dout), lambda i: (0, 0)),
        pl.BlockSpec((1, dout), lambda i: (0, 0)),
        pl.BlockSpec((dout, _T), lambda i: (0, 0)),
        pl.BlockSpec((1, _T), lambda i: (0, 0)),
        pl.BlockSpec((_R, 1), lambda i: (i, 0)),
    ]
    args = [y, st, g.reshape(1, dout), b.reshape(1, dout), lw,
            lb.reshape(1, _T), bat2]
    if zp is not None:
        in_specs.append(pl.BlockSpec((_R, _T), lambda i: (i, 0)))
        args.append(zp)
    return pl.pallas_call(
        functools.partial(_p2_body, gelu_on, zp is not None),
        grid=(grid,),
        in_specs=in_specs,
        out_specs=[
            pl.BlockSpec((_R, dout), lambda i: (i, 0)),
            pl.BlockSpec((_R, _T), lambda i: (i, 0)),
            pl.BlockSpec((_G, _T), lambda i: (0, 0)),
        ],
        out_shape=[
            jax.ShapeDtypeStruct((_N, dout), jnp.float32),
            jax.ShapeDtypeStruct((_N, _T), jnp.float32),
            jax.ShapeDtypeStruct((_G, _T), jnp.float32),
        ],
        compiler_params=_SEQ,
    )(*args)


def kernel(x, edge_index, batch, W0, b0, bn0_g, bn0_b, lin0_W, lin0_b,
           c1_W, c1_b, bn1_g, bn1_b, lin1_W, lin1_b,
           c2_W, c2_b, bn2_g, bn2_b, lin2_W, lin2_b):
    edges3 = edge_index.reshape(2, _EROWS, 128)
    bat2 = batch.reshape(_N, 1)

    y0, st0 = _pass1_first(x, W0)
    h0, z0, seg0 = _pass2(y0, st0, bn0_g, bn0_b, lin0_W, lin0_b, bat2,
                          None, 16, True)

    agg1 = _sc_agg_call(h0, edges3)
    y1, st1 = _pass1_gin(h0, agg1, c1_W, 16)
    h1, z1, seg1 = _pass2(y1, st1, bn1_g, bn1_b, lin1_W, lin1_b, bat2,
                          z0, 16, False)

    agg2 = _sc_agg_call(h1, edges3)
    y2, st2 = _pass1_gin(h1, agg2, c2_W, 8)
    h2, z2, seg2 = _pass2(y2, st2, bn2_g, bn2_b, lin2_W, lin2_b, bat2,
                          z1, 8, False)

    out = seg0 + seg1 + seg2
    return (out, z2, h2)


# speedup vs baseline: 32.7280x; 1.4431x over previous
import functools

import jax
import jax.numpy as jnp
from jax import lax
from jax.experimental import pallas as pl
from jax.experimental.pallas import tpu as pltpu
from jax.experimental.pallas import tpu_sc as plsc

_N = 100000
_F = 128
_E = 3200000
_G = 8
_T = 4

_NC = 2
_NS = 16
_NW = _NC * _NS
_KCH = 8
_EROWS = _E // 128
_RPT = _EROWS // _NW
_RREM = _EROWS % _NW
_PER_T = 6250
_N_ACC = _NS * _PER_T
_ZROWS = 250
_CHUNKS = [(o, _ZROWS) for o in range(0, _PER_T, _ZROWS)]

_R = 2000
_INV_SQRT2 = 0.7071067811865476


def _gelu(v):
    return 0.5 * v * (1.0 + lax.erf(v * _INV_SQRT2))


def _sc_agg_call(h, edges3):
    mesh = plsc.VectorSubcoreMesh(core_axis_name="c", subcore_axis_name="s")

    @functools.partial(
        pl.kernel,
        out_type=jax.ShapeDtypeStruct((_NC, _N_ACC, 16), jnp.float32),
        mesh=mesh,
        scratch_types=[
            pltpu.VMEM((_KCH, 128), jnp.int32),
            pltpu.VMEM((_KCH, 128), jnp.int32),
            pltpu.VMEM((_KCH, 128, 16), jnp.float32),
            pltpu.VMEM((_ZROWS, 16), jnp.float32),
            pltpu.VMEM_SHARED((_N_ACC, 16), jnp.float32),
            pltpu.SemaphoreType.DMA,
        ],
        compiler_params=pltpu.CompilerParams(use_tc_tiling_on_sc=False),
    )
    def body(h_hbm, e_hbm, out_hbm, src_v, dst_v, rows_v, zbuf, acc, sem):
        cid = lax.axis_index("c")
        sid = lax.axis_index("s")
        wid = sid * _NC + cid

        def zrow(i, carry):
            zbuf[i, :] = jnp.zeros((16,), jnp.float32)
            return carry

        lax.fori_loop(0, _ZROWS, zrow, 0)
        for off, sz in _CHUNKS:
            pltpu.sync_copy(zbuf.at[pl.ds(0, sz)],
                            acc.at[pl.ds(sid * _PER_T + off, sz)])
        plsc.subcore_barrier()

        start = wid * _RPT + jnp.minimum(wid, _RREM)
        count = _RPT + (wid < _RREM).astype(jnp.int32)
        nfull = count // _KCH

        def gather_scatter(row0, k):
            pltpu.sync_copy(e_hbm.at[0, pl.ds(row0, k)], src_v.at[pl.ds(0, k)])
            pltpu.sync_copy(e_hbm.at[1, pl.ds(row0, k)], dst_v.at[pl.ds(0, k)])
            copies = [
                pltpu.async_copy(h_hbm.at[src_v.at[j]], rows_v.at[j], sem)
                for j in range(k)
            ]
            for c in copies:
                c.wait()
            for j in range(k):
                pltpu.sync_copy(rows_v.at[j], acc.at[dst_v.at[j]], add=True)

        def step(it, carry):
            gather_scatter(start + it * _KCH, _KCH)
            return carry

        lax.fori_loop(0, nfull, step, 0)

        def tail(it, carry):
            gather_scatter(start + nfull * _KCH + it, 1)
            return carry

        lax.fori_loop(0, count - nfull * _KCH, tail, 0)
        plsc.subcore_barrier()
        for off, sz in _CHUNKS:
            sl = pl.ds(sid * _PER_T + off, sz)
            pltpu.sync_copy(acc.at[sl], out_hbm.at[cid].at[sl])

    return body(h, edges3)


_PROWS = _N // 8
_RP = 1024
_PGRID = -(-_PROWS // _RP)


def _stats_update(i, y, st_ref):
    s = jnp.sum(y, axis=0, keepdims=True)
    sq = jnp.sum(y * y, axis=0, keepdims=True)
    upd = jnp.concatenate([s, sq, jnp.zeros((6, y.shape[1]), jnp.float32)], 0)

    @pl.when(i == 0)
    def _():
        st_ref[...] = jnp.zeros_like(st_ref)

    st_ref[...] += upd


def _p1_first_body(x_ref, w_ref, y_ref, st_ref):
    i = pl.program_id(0)
    y = jnp.dot(x_ref[...], w_ref[...], preferred_element_type=jnp.float32)
    rid = lax.broadcasted_iota(jnp.int32, (y.shape[0], 1), 0) + i * _RP
    y = jnp.where(rid < _PROWS, y, 0.0)
    y_ref[...] = y
    _stats_update(i, y, st_ref)


def _p1_gin_body(h_ref, agga_ref, aggb_ref, w_ref, y_ref, st_ref):
    i = pl.program_id(0)
    hin = h_ref[...] + agga_ref[0] + aggb_ref[0]
    y = jnp.dot(hin, w_ref[...], preferred_element_type=jnp.float32)
    rid = lax.broadcasted_iota(jnp.int32, (y.shape[0], 1), 0) + i * _RP
    y = jnp.where(rid < _PROWS, y, 0.0)
    y_ref[...] = y
    _stats_update(i, y, st_ref)


def _p2_body(gelu_on, has_zp, *refs):
    if has_zp:
        (y_ref, sc_ref, sh_ref, wz_ref, lbz_ref, bat_ref, zp_ref,
         h_ref, z_ref, seg_ref) = refs
    else:
        (y_ref, sc_ref, sh_ref, wz_ref, lbz_ref, bat_ref,
         h_ref, z_ref, seg_ref) = refs
        zp_ref = None
    i = pl.program_id(0)
    h = y_ref[...] * sc_ref[...] + sh_ref[...]
    if gelu_on:
        h = _gelu(h)
    h_ref[...] = h
    z = jnp.dot(h, wz_ref[...], preferred_element_type=jnp.float32) + lbz_ref[...]
    if gelu_on:
        z = _gelu(z)
    if zp_ref is not None:
        z_ref[...] = z + zp_ref[...]
    else:
        z_ref[...] = z
    bat = bat_ref[...]
    rid = lax.broadcasted_iota(jnp.int32, (z.shape[0], 1), 0) + i * _RP
    valid = rid < _PROWS
    rows = [
        jnp.max(jnp.where((bat == g) & valid, z, -jnp.inf), axis=0, keepdims=True)
        for g in range(_G)
    ]
    cur = jnp.concatenate(rows, 0)

    @pl.when(i == 0)
    def _():
        seg_ref[...] = jnp.full_like(seg_ref, -jnp.inf)

    seg_ref[...] = jnp.maximum(seg_ref[...], cur)


_SEQ = pltpu.CompilerParams(dimension_semantics=("arbitrary",))


def _pass1_first(x2, wbig):
    return pl.pallas_call(
        _p1_first_body,
        grid=(_PGRID,),
        in_specs=[
            pl.BlockSpec((_RP, 8 * _F), lambda i: (i, 0)),
            pl.BlockSpec((8 * _F, 128), lambda i: (0, 0)),
        ],
        out_specs=[
            pl.BlockSpec((_RP, 128), lambda i: (i, 0)),
            pl.BlockSpec((8, 128), lambda i: (0, 0)),
        ],
        out_shape=[
            jax.ShapeDtypeStruct((_PROWS, 128), jnp.float32),
            jax.ShapeDtypeStruct((8, 128), jnp.float32),
        ],
        compiler_params=_SEQ,
    )(x2, wbig)


def _pass1_gin(hp, aggp, wbig, dout):
    return pl.pallas_call(
        _p1_gin_body,
        grid=(_PGRID,),
        in_specs=[
            pl.BlockSpec((_RP, 128), lambda i: (i, 0)),
            pl.BlockSpec((1, _RP, 128), lambda i: (0, i, 0)),
            pl.BlockSpec((1, _RP, 128), lambda i: (1, i, 0)),
            pl.BlockSpec((128, dout), lambda i: (0, 0)),
        ],
        out_specs=[
            pl.BlockSpec((_RP, dout), lambda i: (i, 0)),
            pl.BlockSpec((8, dout), lambda i: (0, 0)),
        ],
        out_shape=[
            jax.ShapeDtypeStruct((_PROWS, dout), jnp.float32),
            jax.ShapeDtypeStruct((8, dout), jnp.float32),
        ],
        compiler_params=_SEQ,
    )(hp, aggp, aggp, wbig)


def _pass2(yp, sc, sh, wz, lbz, batrep, zp, dout, gelu_on):
    in_specs = [
        pl.BlockSpec((_RP, dout), lambda i: (i, 0)),
        pl.BlockSpec((1, dout), lambda i: (0, 0)),
        pl.BlockSpec((1, dout), lambda i: (0, 0)),
        pl.BlockSpec((dout, 32), lambda i: (0, 0)),
        pl.BlockSpec((1, 32), lambda i: (0, 0)),
        pl.BlockSpec((_RP, 32), lambda i: (i, 0)),
    ]
    args = [yp, sc, sh, wz, lbz, batrep]
    if zp is not None:
        in_specs.append(pl.BlockSpec((_RP, 32), lambda i: (i, 0)))
        args.append(zp)
    return pl.pallas_call(
        functools.partial(_p2_body, gelu_on, zp is not None),
        grid=(_PGRID,),
        in_specs=in_specs,
        out_specs=[
            pl.BlockSpec((_RP, dout), lambda i: (i, 0)),
            pl.BlockSpec((_RP, 32), lambda i: (i, 0)),
            pl.BlockSpec((_G, 32), lambda i: (0, 0)),
        ],
        out_shape=[
            jax.ShapeDtypeStruct((_PROWS, dout), jnp.float32),
            jax.ShapeDtypeStruct((_PROWS, 32), jnp.float32),
            jax.ShapeDtypeStruct((_G, 32), jnp.float32),
        ],
        compiler_params=_SEQ,
    )(*args)


def _bn_coeffs(st, g, b, dsub):
    s = st[0].reshape(8, dsub).sum(0)
    sq = st[1].reshape(8, dsub).sum(0)
    m = s * (1.0 / _N)
    v = sq * (1.0 / _N) - m * m
    scale = g * lax.rsqrt(v + 1e-5)
    shift = b - m * scale
    return jnp.tile(scale, 8).reshape(1, 8 * dsub), jnp.tile(shift, 8).reshape(1, 8 * dsub)


def kernel(x, edge_index, batch, W0, b0, bn0_g, bn0_b, lin0_W, lin0_b,
           c1_W, c1_b, bn1_g, bn1_b, lin1_W, lin1_b,
           c2_W, c2_b, bn2_g, bn2_b, lin2_W, lin2_b):
    edges3 = edge_index.reshape(2, _EROWS, 128)
    batrep = jnp.repeat(batch, 4).reshape(_PROWS, 32)
    eye8 = jnp.eye(8, dtype=jnp.float32)

    y0, st0 = _pass1_first(x.reshape(_PROWS, 8 * _F), jnp.kron(eye8, W0))
    sc0, sh0 = _bn_coeffs(st0, bn0_g, bn0_b, 16)
    h0, z0, seg0 = _pass2(y0, sc0, sh0, jnp.kron(eye8, lin0_W),
                          jnp.tile(lin0_b, 8).reshape(1, 32), batrep,
                          None, 128, True)

    agg1 = _sc_agg_call(h0.reshape(_N, 16), edges3).reshape(2, _PROWS, 128)
    y1, st1 = _pass1_gin(h0, agg1, jnp.kron(eye8, c1_W), 128)
    sc1, sh1 = _bn_coeffs(st1, bn1_g, bn1_b, 16)
    h1, z1, seg1 = _pass2(y1, sc1, sh1, jnp.kron(eye8, lin1_W),
                          jnp.tile(lin1_b, 8).reshape(1, 32), batrep,
                          z0, 128, False)

    agg2 = _sc_agg_call(h1.reshape(_N, 16), edges3).reshape(2, _PROWS, 128)
    y2, st2 = _pass1_gin(h1, agg2, jnp.kron(eye8, c2_W), 64)
    sc2, sh2 = _bn_coeffs(st2, bn2_g, bn2_b, 8)
    h2, z2, seg2 = _pass2(y2, sc2, sh2, jnp.kron(eye8, lin2_W),
                          jnp.tile(lin2_b, 8).reshape(1, 32), batrep,
                          z1, 64, False)

    segf = lambda s: s.reshape(_G, 8, _T).max(1)
    out = segf(seg0) + segf(seg1) + segf(seg2)
    return (out, z2.reshape(_N, _T), h2.reshape(_N, 8))


# speedup vs baseline: 44.6154x; 1.3632x over previous
import functools

import jax
import jax.numpy as jnp
from jax import lax
from jax.experimental import pallas as pl
from jax.experimental.pallas import tpu as pltpu
from jax.experimental.pallas import tpu_sc as plsc

_N = 100000
_F = 128
_E = 3200000
_G = 8
_T = 4

_NC = 2
_NS = 16
_NW = _NC * _NS
_KCH = 4
_EROWS = _E // 128
_RPT = _EROWS // _NW
_RREM = _EROWS % _NW
_NFULL = _RPT // _KCH
_PER_T = 6250
_N_ACC = _NS * _PER_T
_ZROWS = 250
_CHUNKS = [(o, _ZROWS) for o in range(0, _PER_T, _ZROWS)]

_R = 2000
_INV_SQRT2 = 0.7071067811865476


def _gelu(v):
    return 0.5 * v * (1.0 + lax.erf(v * _INV_SQRT2))


def _sc_agg_call(h, edges3):
    mesh = plsc.VectorSubcoreMesh(core_axis_name="c", subcore_axis_name="s")

    @functools.partial(
        pl.kernel,
        out_type=jax.ShapeDtypeStruct((_NC, _N_ACC, 16), jnp.float32),
        mesh=mesh,
        scratch_types=[
            pltpu.VMEM((_KCH, 128), jnp.int32),
            pltpu.VMEM((_KCH, 128), jnp.int32),
            pltpu.VMEM((_KCH, 128), jnp.int32),
            pltpu.VMEM((_KCH, 128), jnp.int32),
            pltpu.VMEM((_KCH, 128, 16), jnp.float32),
            pltpu.VMEM((_KCH, 128, 16), jnp.float32),
            pltpu.VMEM((_ZROWS, 16), jnp.float32),
            pltpu.VMEM_SHARED((_N_ACC, 16), jnp.float32),
            pltpu.SemaphoreType.DMA,
            pltpu.SemaphoreType.DMA,
            pltpu.SemaphoreType.DMA,
            pltpu.SemaphoreType.DMA,
        ],
        compiler_params=pltpu.CompilerParams(use_tc_tiling_on_sc=False),
    )
    def body(h_hbm, e_hbm, out_hbm, src0, dst0, src1, dst1, rows0, rows1,
             zbuf, acc, sg0, sg1, ss0, ss1):
        cid = lax.axis_index("c")
        sid = lax.axis_index("s")
        wid = sid * _NC + cid

        def zrow(i, carry):
            zbuf[i, :] = jnp.zeros((16,), jnp.float32)
            return carry

        lax.fori_loop(0, _ZROWS, zrow, 0)
        for off, sz in _CHUNKS:
            pltpu.sync_copy(zbuf.at[pl.ds(0, sz)],
                            acc.at[pl.ds(sid * _PER_T + off, sz)])
        plsc.subcore_barrier()

        start = wid * _RPT + jnp.minimum(wid, _RREM)
        count = _RPT + (wid < _RREM).astype(jnp.int32)
        bufs = ((src0, dst0, rows0, sg0, ss0), (src1, dst1, rows1, sg1, ss1))

        def load_idx(b, row0):
            pltpu.sync_copy(e_hbm.at[0, pl.ds(row0, _KCH)], bufs[b][0])
            pltpu.sync_copy(e_hbm.at[1, pl.ds(row0, _KCH)], bufs[b][1])

        def fire_g(b):
            for j in range(_KCH):
                pltpu.async_copy(h_hbm.at[bufs[b][0].at[j]],
                                 bufs[b][2].at[j], bufs[b][3])

        def wait_g(b):
            for j in range(_KCH):
                pltpu.make_async_copy(h_hbm.at[bufs[b][0].at[j]],
                                      bufs[b][2].at[j], bufs[b][3]).wait()

        def fire_s(b):
            for j in range(_KCH):
                pltpu.async_copy(bufs[b][2].at[j], acc.at[bufs[b][1].at[j]],
                                 bufs[b][4], add=True)

        def wait_s(b):
            for j in range(_KCH):
                pltpu.make_async_copy(bufs[b][2].at[j],
                                      acc.at[bufs[b][1].at[j]],
                                      bufs[b][4]).wait()

        load_idx(0, start)
        fire_g(0)

        def pair(it2, carry):
            it = 2 * it2
            load_idx(1, start + (it + 1) * _KCH)

            @pl.when(it2 > 0)
            def _():
                wait_s(1)

            fire_g(1)
            wait_g(0)
            fire_s(0)
            load_idx(0, start + (it + 2) * _KCH)
            wait_s(0)
            fire_g(0)
            wait_g(1)
            fire_s(1)
            return carry

        lax.fori_loop(0, (_NFULL - 1) // 2, pair, 0)
        wait_g(0)
        fire_s(0)
        wait_s(1)
        wait_s(0)

        def tail(it, carry):
            row = start + _NFULL * _KCH + it
            pltpu.sync_copy(e_hbm.at[0, pl.ds(row, 1)], src0.at[pl.ds(0, 1)])
            pltpu.sync_copy(e_hbm.at[1, pl.ds(row, 1)], dst0.at[pl.ds(0, 1)])
            pltpu.async_copy(h_hbm.at[src0.at[0]], rows0.at[0], sg0).wait()
            pltpu.sync_copy(rows0.at[0], acc.at[dst0.at[0]], add=True)
            return carry

        lax.fori_loop(0, count - _NFULL * _KCH, tail, 0)
        plsc.subcore_barrier()
        for off, sz in _CHUNKS:
            sl = pl.ds(sid * _PER_T + off, sz)
            pltpu.sync_copy(acc.at[sl], out_hbm.at[cid].at[sl])

    return body(h, edges3)


_PROWS = _N // 8
_RP = 1024
_PGRID = -(-_PROWS // _RP)


def _stats_update(i, y, st_ref):
    s = jnp.sum(y, axis=0, keepdims=True)
    sq = jnp.sum(y * y, axis=0, keepdims=True)
    upd = jnp.concatenate([s, sq, jnp.zeros((6, y.shape[1]), jnp.float32)], 0)

    @pl.when(i == 0)
    def _():
        st_ref[...] = jnp.zeros_like(st_ref)

    st_ref[...] += upd


def _p1_first_body(x_ref, w_ref, y_ref, st_ref):
    i = pl.program_id(0)
    y = jnp.dot(x_ref[...], w_ref[...], preferred_element_type=jnp.float32)
    rid = lax.broadcasted_iota(jnp.int32, (y.shape[0], 1), 0) + i * _RP
    y = jnp.where(rid < _PROWS, y, 0.0)
    y_ref[...] = y
    _stats_update(i, y, st_ref)


def _p1_gin_body(h_ref, agga_ref, aggb_ref, w_ref, y_ref, st_ref):
    i = pl.program_id(0)
    hin = h_ref[...] + agga_ref[0] + aggb_ref[0]
    y = jnp.dot(hin, w_ref[...], preferred_element_type=jnp.float32)
    rid = lax.broadcasted_iota(jnp.int32, (y.shape[0], 1), 0) + i * _RP
    y = jnp.where(rid < _PROWS, y, 0.0)
    y_ref[...] = y
    _stats_update(i, y, st_ref)


def _p2_body(gelu_on, has_zp, *refs):
    if has_zp:
        (y_ref, sc_ref, sh_ref, wz_ref, lbz_ref, bat_ref, zp_ref,
         h_ref, z_ref, seg_ref) = refs
    else:
        (y_ref, sc_ref, sh_ref, wz_ref, lbz_ref, bat_ref,
         h_ref, z_ref, seg_ref) = refs
        zp_ref = None
    i = pl.program_id(0)
    h = y_ref[...] * sc_ref[...] + sh_ref[...]
    if gelu_on:
        h = _gelu(h)
    h_ref[...] = h
    z = jnp.dot(h, wz_ref[...], preferred_element_type=jnp.float32) + lbz_ref[...]
    if gelu_on:
        z = _gelu(z)
    if zp_ref is not None:
        z_ref[...] = z + zp_ref[...]
    else:
        z_ref[...] = z
    bat = bat_ref[...]
    rid = lax.broadcasted_iota(jnp.int32, (z.shape[0], 1), 0) + i * _RP
    valid = rid < _PROWS
    rows = [
        jnp.max(jnp.where((bat == g) & valid, z, -jnp.inf), axis=0, keepdims=True)
        for g in range(_G)
    ]
    cur = jnp.concatenate(rows, 0)

    @pl.when(i == 0)
    def _():
        seg_ref[...] = jnp.full_like(seg_ref, -jnp.inf)

    seg_ref[...] = jnp.maximum(seg_ref[...], cur)


_SEQ = pltpu.CompilerParams(dimension_semantics=("arbitrary",))


def _pass1_first(x2, wbig):
    return pl.pallas_call(
        _p1_first_body,
        grid=(_PGRID,),
        in_specs=[
            pl.BlockSpec((_RP, 8 * _F), lambda i: (i, 0)),
            pl.BlockSpec((8 * _F, 128), lambda i: (0, 0)),
        ],
        out_specs=[
            pl.BlockSpec((_RP, 128), lambda i: (i, 0)),
            pl.BlockSpec((8, 128), lambda i: (0, 0)),
        ],
        out_shape=[
            jax.ShapeDtypeStruct((_PROWS, 128), jnp.float32),
            jax.ShapeDtypeStruct((8, 128), jnp.float32),
        ],
        compiler_params=_SEQ,
    )(x2, wbig)


def _pass1_gin(hp, aggp, wbig, dout):
    return pl.pallas_call(
        _p1_gin_body,
        grid=(_PGRID,),
        in_specs=[
            pl.BlockSpec((_RP, 128), lambda i: (i, 0)),
            pl.BlockSpec((1, _RP, 128), lambda i: (0, i, 0)),
            pl.BlockSpec((1, _RP, 128), lambda i: (1, i, 0)),
            pl.BlockSpec((128, dout), lambda i: (0, 0)),
        ],
        out_specs=[
            pl.BlockSpec((_RP, dout), lambda i: (i, 0)),
            pl.BlockSpec((8, dout), lambda i: (0, 0)),
        ],
        out_shape=[
            jax.ShapeDtypeStruct((_PROWS, dout), jnp.float32),
            jax.ShapeDtypeStruct((8, dout), jnp.float32),
        ],
        compiler_params=_SEQ,
    )(hp, aggp, aggp, wbig)


def _pass2(yp, sc, sh, wz, lbz, batrep, zp, dout, gelu_on):
    in_specs = [
        pl.BlockSpec((_RP, dout), lambda i: (i, 0)),
        pl.BlockSpec((1, dout), lambda i: (0, 0)),
        pl.BlockSpec((1, dout), lambda i: (0, 0)),
        pl.BlockSpec((dout, 32), lambda i: (0, 0)),
        pl.BlockSpec((1, 32), lambda i: (0, 0)),
        pl.BlockSpec((_RP, 32), lambda i: (i, 0)),
    ]
    args = [yp, sc, sh, wz, lbz, batrep]
    if zp is not None:
        in_specs.append(pl.BlockSpec((_RP, 32), lambda i: (i, 0)))
        args.append(zp)
    return pl.pallas_call(
        functools.partial(_p2_body, gelu_on, zp is not None),
        grid=(_PGRID,),
        in_specs=in_specs,
        out_specs=[
            pl.BlockSpec((_RP, dout), lambda i: (i, 0)),
            pl.BlockSpec((_RP, 32), lambda i: (i, 0)),
            pl.BlockSpec((_G, 32), lambda i: (0, 0)),
        ],
        out_shape=[
            jax.ShapeDtypeStruct((_PROWS, dout), jnp.float32),
            jax.ShapeDtypeStruct((_PROWS, 32), jnp.float32),
            jax.ShapeDtypeStruct((_G, 32), jnp.float32),
        ],
        compiler_params=_SEQ,
    )(*args)


def _bn_coeffs(st, g, b, dsub):
    s = st[0].reshape(8, dsub).sum(0)
    sq = st[1].reshape(8, dsub).sum(0)
    m = s * (1.0 / _N)
    v = sq * (1.0 / _N) - m * m
    scale = g * lax.rsqrt(v + 1e-5)
    shift = b - m * scale
    return jnp.tile(scale, 8).reshape(1, 8 * dsub), jnp.tile(shift, 8).reshape(1, 8 * dsub)


def kernel(x, edge_index, batch, W0, b0, bn0_g, bn0_b, lin0_W, lin0_b,
           c1_W, c1_b, bn1_g, bn1_b, lin1_W, lin1_b,
           c2_W, c2_b, bn2_g, bn2_b, lin2_W, lin2_b):
    edges3 = edge_index.reshape(2, _EROWS, 128)
    batrep = jnp.repeat(batch, 4).reshape(_PROWS, 32)
    eye8 = jnp.eye(8, dtype=jnp.float32)

    y0, st0 = _pass1_first(x.reshape(_PROWS, 8 * _F), jnp.kron(eye8, W0))
    sc0, sh0 = _bn_coeffs(st0, bn0_g, bn0_b, 16)
    h0, z0, seg0 = _pass2(y0, sc0, sh0, jnp.kron(eye8, lin0_W),
                          jnp.tile(lin0_b, 8).reshape(1, 32), batrep,
                          None, 128, True)

    agg1 = _sc_agg_call(h0.reshape(_N, 16), edges3).reshape(2, _PROWS, 128)
    y1, st1 = _pass1_gin(h0, agg1, jnp.kron(eye8, c1_W), 128)
    sc1, sh1 = _bn_coeffs(st1, bn1_g, bn1_b, 16)
    h1, z1, seg1 = _pass2(y1, sc1, sh1, jnp.kron(eye8, lin1_W),
                          jnp.tile(lin1_b, 8).reshape(1, 32), batrep,
                          z0, 128, False)

    agg2 = _sc_agg_call(h1.reshape(_N, 16), edges3).reshape(2, _PROWS, 128)
    y2, st2 = _pass1_gin(h1, agg2, jnp.kron(eye8, c2_W), 64)
    sc2, sh2 = _bn_coeffs(st2, bn2_g, bn2_b, 8)
    h2, z2, seg2 = _pass2(y2, sc2, sh2, jnp.kron(eye8, lin2_W),
                          jnp.tile(lin2_b, 8).reshape(1, 32), batrep,
                          z1, 64, False)

    segf = lambda s: s.reshape(_G, 8, _T).max(1)
    out = segf(seg0) + segf(seg1) + segf(seg2)
    return (out, z2.reshape(_N, _T), h2.reshape(_N, 8))


# speedup vs baseline: 44.8185x; 1.0046x over previous
import functools

import jax
import jax.numpy as jnp
from jax import lax
from jax.experimental import pallas as pl
from jax.experimental.pallas import tpu as pltpu
from jax.experimental.pallas import tpu_sc as plsc

_N = 100000
_F = 128
_E = 3200000
_G = 8
_T = 4

_NC = 2
_NS = 16
_NW = _NC * _NS
_KCH = 4
_EROWS = _E // 128
_RPT = _EROWS // _NW
_RREM = _EROWS % _NW
_NFULL = _RPT // _KCH
_PER_T = 6250
_N_ACC = _NS * _PER_T
_ZROWS = 250
_CHUNKS = [(o, _ZROWS) for o in range(0, _PER_T, _ZROWS)]

_R = 2000
_INV_SQRT2 = 0.7071067811865476


def _gelu(v):
    return 0.5 * v * (1.0 + lax.erf(v * _INV_SQRT2))


def _sc_agg_call(h, edges3):
    mesh = plsc.VectorSubcoreMesh(core_axis_name="c", subcore_axis_name="s")

    @functools.partial(
        pl.kernel,
        out_type=jax.ShapeDtypeStruct((_NC, _N_ACC, 16), jnp.float32),
        mesh=mesh,
        scratch_types=[
            pltpu.VMEM((_KCH, 128), jnp.int32),
            pltpu.VMEM((_KCH, 128), jnp.int32),
            pltpu.VMEM((_KCH, 128), jnp.int32),
            pltpu.VMEM((_KCH, 128), jnp.int32),
            pltpu.VMEM((_KCH, 128, 16), jnp.float32),
            pltpu.VMEM((_KCH, 128, 16), jnp.float32),
            pltpu.VMEM((_ZROWS, 16), jnp.float32),
            pltpu.VMEM_SHARED((_N_ACC, 16), jnp.float32),
            pltpu.SemaphoreType.DMA,
            pltpu.SemaphoreType.DMA,
            pltpu.SemaphoreType.DMA,
            pltpu.SemaphoreType.DMA,
        ],
        compiler_params=pltpu.CompilerParams(use_tc_tiling_on_sc=False),
    )
    def body(h_hbm, e_hbm, out_hbm, src0, dst0, src1, dst1, rows0, rows1,
             zbuf, acc, sg0, sg1, ss0, ss1):
        cid = lax.axis_index("c")
        sid = lax.axis_index("s")
        wid = sid * _NC + cid

        def zrow(i, carry):
            zbuf[i, :] = jnp.zeros((16,), jnp.float32)
            return carry

        lax.fori_loop(0, _ZROWS, zrow, 0)
        for off, sz in _CHUNKS:
            pltpu.sync_copy(zbuf.at[pl.ds(0, sz)],
                            acc.at[pl.ds(sid * _PER_T + off, sz)])
        plsc.subcore_barrier()

        start = wid * _RPT + jnp.minimum(wid, _RREM)
        count = _RPT + (wid < _RREM).astype(jnp.int32)
        bufs = ((src0, dst0, rows0, sg0, ss0), (src1, dst1, rows1, sg1, ss1))

        def load_idx(b, row0):
            pltpu.sync_copy(e_hbm.at[0, pl.ds(row0, _KCH)], bufs[b][0])
            pltpu.sync_copy(e_hbm.at[1, pl.ds(row0, _KCH)], bufs[b][1])

        def fire_g(b):
            for j in range(_KCH):
                pltpu.async_copy(h_hbm.at[bufs[b][0].at[j]],
                                 bufs[b][2].at[j], bufs[b][3])

        def wait_g(b):
            for j in range(_KCH):
                pltpu.make_async_copy(h_hbm.at[bufs[b][0].at[j]],
                                      bufs[b][2].at[j], bufs[b][3]).wait()

        def fire_s(b):
            for j in range(_KCH):
                pltpu.async_copy(bufs[b][2].at[j], acc.at[bufs[b][1].at[j]],
                                 bufs[b][4], add=True)

        def wait_s(b):
            for j in range(_KCH):
                pltpu.make_async_copy(bufs[b][2].at[j],
                                      acc.at[bufs[b][1].at[j]],
                                      bufs[b][4]).wait()

        load_idx(0, start)
        fire_g(0)

        def pair(it2, carry):
            it = 2 * it2
            load_idx(1, start + (it + 1) * _KCH)

            @pl.when(it2 > 0)
            def _():
                wait_s(1)

            fire_g(1)
            wait_g(0)
            fire_s(0)
            load_idx(0, start + (it + 2) * _KCH)
            wait_s(0)
            fire_g(0)
            wait_g(1)
            fire_s(1)
            return carry

        lax.fori_loop(0, (_NFULL - 1) // 2, pair, 0)
        wait_g(0)
        fire_s(0)
        wait_s(1)
        wait_s(0)

        def tail(it, carry):
            row = start + _NFULL * _KCH + it
            pltpu.sync_copy(e_hbm.at[0, pl.ds(row, 1)], src0.at[pl.ds(0, 1)])
            pltpu.sync_copy(e_hbm.at[1, pl.ds(row, 1)], dst0.at[pl.ds(0, 1)])
            pltpu.async_copy(h_hbm.at[src0.at[0]], rows0.at[0], sg0).wait()
            pltpu.sync_copy(rows0.at[0], acc.at[dst0.at[0]], add=True)
            return carry

        lax.fori_loop(0, count - _NFULL * _KCH, tail, 0)
        plsc.subcore_barrier()
        for off, sz in _CHUNKS:
            sl = pl.ds(sid * _PER_T + off, sz)
            pltpu.sync_copy(acc.at[sl], out_hbm.at[cid].at[sl])

    return body(h, edges3)


_PROWS = _N // 8
_RP = 1024
_PGRID = -(-_PROWS // _RP)


def _stats_update(i, y, st_ref):
    s = jnp.sum(y, axis=0, keepdims=True)
    sq = jnp.sum(y * y, axis=0, keepdims=True)
    upd = jnp.concatenate([s, sq, jnp.zeros((6, y.shape[1]), jnp.float32)], 0)

    @pl.when(i == 0)
    def _():
        st_ref[...] = jnp.zeros_like(st_ref)

    st_ref[...] += upd


def _p1_first_body(x_ref, w_ref, y_ref, st_ref):
    i = pl.program_id(0)
    y = jnp.dot(x_ref[...], w_ref[...], preferred_element_type=jnp.float32)
    rid = lax.broadcasted_iota(jnp.int32, (y.shape[0], 1), 0) + i * (8 * _RP)
    y = jnp.where(rid < _N, y, 0.0)
    _stats_update(i, y, st_ref)
    y_ref[...] = y


def _p1_gin_body(h_ref, agga_ref, aggb_ref, w_ref, y_ref, st_ref):
    i = pl.program_id(0)
    hin = h_ref[...] + agga_ref[0] + aggb_ref[0]
    y = jnp.dot(hin, w_ref[...], preferred_element_type=jnp.float32)
    rid = lax.broadcasted_iota(jnp.int32, (y.shape[0], 1), 0) + i * _RP
    y = jnp.where(rid < _PROWS, y, 0.0)
    y_ref[...] = y
    _stats_update(i, y, st_ref)


def _p2_body(gelu_on, has_zp, *refs):
    if has_zp:
        (y_ref, sc_ref, sh_ref, wz_ref, lbz_ref, bat_ref, zp_ref,
         h_ref, z_ref, seg_ref) = refs
    else:
        (y_ref, sc_ref, sh_ref, wz_ref, lbz_ref, bat_ref,
         h_ref, z_ref, seg_ref) = refs
        zp_ref = None
    i = pl.program_id(0)
    h = y_ref[...] * sc_ref[...] + sh_ref[...]
    if gelu_on:
        h = _gelu(h)
    h_ref[...] = h
    z = jnp.dot(h, wz_ref[...], preferred_element_type=jnp.float32) + lbz_ref[...]
    if gelu_on:
        z = _gelu(z)
    z_ref[...] = z + zp_ref[...] if zp_ref is not None else z
    bat = bat_ref[...]
    rid = lax.broadcasted_iota(jnp.int32, (z.shape[0], 1), 0) + i * _RP
    valid = rid < _PROWS
    rows = [
        jnp.max(jnp.where((bat == g) & valid, z, -jnp.inf), axis=0, keepdims=True)
        for g in range(_G)
    ]
    cur = jnp.concatenate(rows, 0)

    @pl.when(i == 0)
    def _():
        seg_ref[...] = jnp.full_like(seg_ref, -jnp.inf)

    seg_ref[...] = jnp.maximum(seg_ref[...], cur)


_SEQ = pltpu.CompilerParams(dimension_semantics=("arbitrary",))


def _pass1_first(x, w):
    return pl.pallas_call(
        _p1_first_body,
        grid=(_PGRID,),
        in_specs=[
            pl.BlockSpec((8 * _RP, _F), lambda i: (i, 0)),
            pl.BlockSpec((_F, 16), lambda i: (0, 0)),
        ],
        out_specs=[
            pl.BlockSpec((8 * _RP, 16), lambda i: (i, 0)),
            pl.BlockSpec((8, 16), lambda i: (0, 0)),
        ],
        out_shape=[
            jax.ShapeDtypeStruct((_N, 16), jnp.float32),
            jax.ShapeDtypeStruct((8, 16), jnp.float32),
        ],
        compiler_params=_SEQ,
    )(x, w)


def _pass1_gin(hp, aggp, wbig, dout):
    return pl.pallas_call(
        _p1_gin_body,
        grid=(_PGRID,),
        in_specs=[
            pl.BlockSpec((_RP, 128), lambda i: (i, 0)),
            pl.BlockSpec((1, _RP, 128), lambda i: (0, i, 0)),
            pl.BlockSpec((1, _RP, 128), lambda i: (1, i, 0)),
            pl.BlockSpec((128, dout), lambda i: (0, 0)),
        ],
        out_specs=[
            pl.BlockSpec((_RP, dout), lambda i: (i, 0)),
            pl.BlockSpec((8, dout), lambda i: (0, 0)),
        ],
        out_shape=[
            jax.ShapeDtypeStruct((_PROWS, dout), jnp.float32),
            jax.ShapeDtypeStruct((8, dout), jnp.float32),
        ],
        compiler_params=_SEQ,
    )(hp, aggp, aggp, wbig)


def _pass2(yp, sc, sh, wz, lbz, batrep, zp, dout, gelu_on):
    in_specs = [
        pl.BlockSpec((_RP, dout), lambda i: (i, 0)),
        pl.BlockSpec((1, dout), lambda i: (0, 0)),
        pl.BlockSpec((1, dout), lambda i: (0, 0)),
        pl.BlockSpec((dout, 32), lambda i: (0, 0)),
        pl.BlockSpec((1, 32), lambda i: (0, 0)),
        pl.BlockSpec((_RP, 32), lambda i: (i, 0)),
    ]
    args = [yp, sc, sh, wz, lbz, batrep]
    if zp is not None:
        in_specs.append(pl.BlockSpec((_RP, 32), lambda i: (i, 0)))
        args.append(zp)
    out_specs = [
        pl.BlockSpec((_RP, dout), lambda i: (i, 0)),
        pl.BlockSpec((_RP, 32), lambda i: (i, 0)),
        pl.BlockSpec((_G, 32), lambda i: (0, 0)),
    ]
    out_shape = [
        jax.ShapeDtypeStruct((_PROWS, dout), jnp.float32),
        jax.ShapeDtypeStruct((_PROWS, 32), jnp.float32),
        jax.ShapeDtypeStruct((_G, 32), jnp.float32),
    ]
    return pl.pallas_call(
        functools.partial(_p2_body, gelu_on, zp is not None),
        grid=(_PGRID,),
        in_specs=in_specs,
        out_specs=out_specs,
        out_shape=out_shape,
        compiler_params=_SEQ,
    )(*args)


def _bn_coeffs(st, g, b, dsub, folds=8):
    s = st[0].reshape(folds, dsub).sum(0)
    sq = st[1].reshape(folds, dsub).sum(0)
    m = s * (1.0 / _N)
    v = sq * (1.0 / _N) - m * m
    scale = g * lax.rsqrt(v + 1e-5)
    shift = b - m * scale
    return jnp.tile(scale, 8).reshape(1, 8 * dsub), jnp.tile(shift, 8).reshape(1, 8 * dsub)


def kernel(x, edge_index, batch, W0, b0, bn0_g, bn0_b, lin0_W, lin0_b,
           c1_W, c1_b, bn1_g, bn1_b, lin1_W, lin1_b,
           c2_W, c2_b, bn2_g, bn2_b, lin2_W, lin2_b):
    edges3 = edge_index.reshape(2, _EROWS, 128)
    batrep = jnp.repeat(batch, 4).reshape(_PROWS, 32)
    eye8 = jnp.eye(8, dtype=jnp.float32)

    y0, st0 = _pass1_first(x, W0)
    y0 = y0.reshape(_PROWS, 128)
    sc0, sh0 = _bn_coeffs(st0, bn0_g, bn0_b, 16, folds=1)
    h0, z0, seg0 = _pass2(y0, sc0, sh0, jnp.kron(eye8, lin0_W),
                          jnp.tile(lin0_b, 8).reshape(1, 32), batrep,
                          None, 128, True)

    agg1 = _sc_agg_call(h0.reshape(_N, 16), edges3).reshape(2, _PROWS, 128)
    y1, st1 = _pass1_gin(h0, agg1, jnp.kron(eye8, c1_W), 128)
    sc1, sh1 = _bn_coeffs(st1, bn1_g, bn1_b, 16)
    h1, z1, seg1 = _pass2(y1, sc1, sh1, jnp.kron(eye8, lin1_W),
                          jnp.tile(lin1_b, 8).reshape(1, 32), batrep,
                          z0, 128, False)

    agg2 = _sc_agg_call(h1.reshape(_N, 16), edges3).reshape(2, _PROWS, 128)
    y2, st2 = _pass1_gin(h1, agg2, jnp.kron(eye8, c2_W), 64)
    sc2, sh2 = _bn_coeffs(st2, bn2_g, bn2_b, 8)
    h2, z2, seg2 = _pass2(y2, sc2, sh2, jnp.kron(eye8, lin2_W),
                          jnp.tile(lin2_b, 8).reshape(1, 32), batrep,
                          z1, 64, False)

    segf = lambda s: s.reshape(_G, 8, _T).max(1)
    out = segf(seg0) + segf(seg1) + segf(seg2)
    return (out, z2.reshape(_N, _T), h2.reshape(_N, 8))


# speedup vs baseline: 46.6059x; 1.0399x over previous
import functools

import jax
import jax.numpy as jnp
from jax import lax
from jax.experimental import pallas as pl
from jax.experimental.pallas import tpu as pltpu
from jax.experimental.pallas import tpu_sc as plsc

_N = 100000
_F = 128
_E = 3200000
_G = 8
_T = 4

_NC = 2
_NS = 16
_NW = _NC * _NS
_KCH = 4
_EROWS = _E // 128
_RPT = _EROWS // _NW
_RREM = _EROWS % _NW
_NFULL = _RPT // _KCH
_PER_T = 6250
_N_ACC = _NS * _PER_T
_ZROWS = 250
_CHUNKS = [(o, _ZROWS) for o in range(0, _PER_T, _ZROWS)]

_R = 2000
_INV_SQRT2 = 0.7071067811865476


def _gelu(v):
    return 0.5 * v * (1.0 + lax.erf(v * _INV_SQRT2))


def _sc_agg_call(h, edges3):
    mesh = plsc.VectorSubcoreMesh(core_axis_name="c", subcore_axis_name="s")

    @functools.partial(
        pl.kernel,
        out_type=jax.ShapeDtypeStruct((_NC, _N_ACC, 16), jnp.float32),
        mesh=mesh,
        scratch_types=[
            pltpu.VMEM((_KCH, 128), jnp.int32),
            pltpu.VMEM((_KCH, 128), jnp.int32),
            pltpu.VMEM((_KCH, 128), jnp.int32),
            pltpu.VMEM((_KCH, 128), jnp.int32),
            pltpu.VMEM((_KCH, 128, 16), jnp.float32),
            pltpu.VMEM((_KCH, 128, 16), jnp.float32),
            pltpu.VMEM((_ZROWS, 16), jnp.float32),
            pltpu.VMEM_SHARED((_N_ACC, 16), jnp.float32),
            pltpu.SemaphoreType.DMA,
            pltpu.SemaphoreType.DMA,
            pltpu.SemaphoreType.DMA,
            pltpu.SemaphoreType.DMA,
        ],
        compiler_params=pltpu.CompilerParams(use_tc_tiling_on_sc=False),
    )
    def body(h_hbm, e_hbm, out_hbm, src0, dst0, src1, dst1, rows0, rows1,
             zbuf, acc, sg0, sg1, ss0, ss1):
        cid = lax.axis_index("c")
        sid = lax.axis_index("s")
        wid = sid * _NC + cid

        def zrow(i, carry):
            zbuf[i, :] = jnp.zeros((16,), jnp.float32)
            return carry

        lax.fori_loop(0, _ZROWS, zrow, 0)
        for off, sz in _CHUNKS:
            pltpu.sync_copy(zbuf.at[pl.ds(0, sz)],
                            acc.at[pl.ds(sid * _PER_T + off, sz)])
        plsc.subcore_barrier()

        start = wid * _RPT + jnp.minimum(wid, _RREM)
        count = _RPT + (wid < _RREM).astype(jnp.int32)
        bufs = ((src0, dst0, rows0, sg0, ss0), (src1, dst1, rows1, sg1, ss1))

        def load_idx(b, row0):
            pltpu.sync_copy(e_hbm.at[0, pl.ds(row0, _KCH)], bufs[b][0])
            pltpu.sync_copy(e_hbm.at[1, pl.ds(row0, _KCH)], bufs[b][1])

        def fire_g(b):
            for j in range(_KCH):
                pltpu.async_copy(h_hbm.at[bufs[b][0].at[j]],
                                 bufs[b][2].at[j], bufs[b][3])

        def wait_g(b):
            for j in range(_KCH):
                pltpu.make_async_copy(h_hbm.at[bufs[b][0].at[j]],
                                      bufs[b][2].at[j], bufs[b][3]).wait()

        def fire_s(b):
            for j in range(_KCH):
                pltpu.async_copy(bufs[b][2].at[j], acc.at[bufs[b][1].at[j]],
                                 bufs[b][4], add=True)

        def wait_s(b):
            for j in range(_KCH):
                pltpu.make_async_copy(bufs[b][2].at[j],
                                      acc.at[bufs[b][1].at[j]],
                                      bufs[b][4]).wait()

        load_idx(0, start)
        fire_g(0)

        def pair(it2, carry):
            it = 2 * it2
            load_idx(1, start + (it + 1) * _KCH)

            @pl.when(it2 > 0)
            def _():
                wait_s(1)

            fire_g(1)
            wait_g(0)
            fire_s(0)
            load_idx(0, start + (it + 2) * _KCH)
            wait_s(0)
            fire_g(0)
            wait_g(1)
            fire_s(1)
            return carry

        lax.fori_loop(0, (_NFULL - 1) // 2, pair, 0)
        wait_g(0)
        fire_s(0)
        wait_s(1)
        wait_s(0)

        def tail(it, carry):
            row = start + _NFULL * _KCH + it
            pltpu.sync_copy(e_hbm.at[0, pl.ds(row, 1)], src0.at[pl.ds(0, 1)])
            pltpu.sync_copy(e_hbm.at[1, pl.ds(row, 1)], dst0.at[pl.ds(0, 1)])
            pltpu.async_copy(h_hbm.at[src0.at[0]], rows0.at[0], sg0).wait()
            pltpu.sync_copy(rows0.at[0], acc.at[dst0.at[0]], add=True)
            return carry

        lax.fori_loop(0, count - _NFULL * _KCH, tail, 0)
        plsc.subcore_barrier()
        for off, sz in _CHUNKS:
            sl = pl.ds(sid * _PER_T + off, sz)
            pltpu.sync_copy(acc.at[sl], out_hbm.at[cid].at[sl])

    return body(h, edges3)


_PROWS = _N // 8
_RP = 1024
_PGRID = -(-_PROWS // _RP)


def _stats_update(i, y, st_ref):
    s = jnp.sum(y, axis=0, keepdims=True)
    sq = jnp.sum(y * y, axis=0, keepdims=True)
    upd = jnp.concatenate([s, sq, jnp.zeros((6, y.shape[1]), jnp.float32)], 0)

    @pl.when(i == 0)
    def _():
        st_ref[...] = jnp.zeros_like(st_ref)

    st_ref[...] += upd


def _p1_first_body(x_ref, w_ref, y_ref, st_ref):
    i = pl.program_id(0)
    w = w_ref[...]
    parts = [
        jnp.dot(x_ref[:, j, :], w, preferred_element_type=jnp.float32)
        for j in range(8)
    ]
    y = jnp.concatenate(parts, axis=1)
    rid = lax.broadcasted_iota(jnp.int32, (y.shape[0], 1), 0) + i * _RP
    y = jnp.where(rid < _PROWS, y, 0.0)
    _stats_update(i, y, st_ref)
    y_ref[...] = y


def _p1_gin_body(h_ref, agga_ref, aggb_ref, w_ref, y_ref, st_ref):
    i = pl.program_id(0)
    hin = h_ref[...] + agga_ref[0] + aggb_ref[0]
    y = jnp.dot(hin, w_ref[...], preferred_element_type=jnp.float32)
    rid = lax.broadcasted_iota(jnp.int32, (y.shape[0], 1), 0) + i * _RP
    y = jnp.where(rid < _PROWS, y, 0.0)
    y_ref[...] = y
    _stats_update(i, y, st_ref)


def _p2_body(gelu_on, has_zp, *refs):
    if has_zp:
        (y_ref, sc_ref, sh_ref, wz_ref, lbz_ref, bat_ref, zp_ref,
         h_ref, z_ref, seg_ref) = refs
    else:
        (y_ref, sc_ref, sh_ref, wz_ref, lbz_ref, bat_ref,
         h_ref, z_ref, seg_ref) = refs
        zp_ref = None
    i = pl.program_id(0)
    h = y_ref[...] * sc_ref[...] + sh_ref[...]
    if gelu_on:
        h = _gelu(h)
    h_ref[...] = h
    z = jnp.dot(h, wz_ref[...], preferred_element_type=jnp.float32) + lbz_ref[...]
    if gelu_on:
        z = _gelu(z)
    z_ref[...] = z + zp_ref[...] if zp_ref is not None else z
    bat = bat_ref[...]
    rid = lax.broadcasted_iota(jnp.int32, (z.shape[0], 1), 0) + i * _RP
    valid = rid < _PROWS
    rows = [
        jnp.max(jnp.where((bat == g) & valid, z, -jnp.inf), axis=0, keepdims=True)
        for g in range(_G)
    ]
    cur = jnp.concatenate(rows, 0)

    @pl.when(i == 0)
    def _():
        seg_ref[...] = jnp.full_like(seg_ref, -jnp.inf)

    seg_ref[...] = jnp.maximum(seg_ref[...], cur)


_SEQ = pltpu.CompilerParams(dimension_semantics=("arbitrary",))


def _pass1_first(x, w):
    return pl.pallas_call(
        _p1_first_body,
        grid=(_PGRID,),
        in_specs=[
            pl.BlockSpec((_RP, 8, _F), lambda i: (i, 0, 0)),
            pl.BlockSpec((_F, 16), lambda i: (0, 0)),
        ],
        out_specs=[
            pl.BlockSpec((_RP, 128), lambda i: (i, 0)),
            pl.BlockSpec((8, 128), lambda i: (0, 0)),
        ],
        out_shape=[
            jax.ShapeDtypeStruct((_PROWS, 128), jnp.float32),
            jax.ShapeDtypeStruct((8, 128), jnp.float32),
        ],
        compiler_params=_SEQ,
    )(x.reshape(_PROWS, 8, _F), w)


def _pass1_gin(hp, aggp, wbig, dout):
    return pl.pallas_call(
        _p1_gin_body,
        grid=(_PGRID,),
        in_specs=[
            pl.BlockSpec((_RP, 128), lambda i: (i, 0)),
            pl.BlockSpec((1, _RP, 128), lambda i: (0, i, 0)),
            pl.BlockSpec((1, _RP, 128), lambda i: (1, i, 0)),
            pl.BlockSpec((128, dout), lambda i: (0, 0)),
        ],
        out_specs=[
            pl.BlockSpec((_RP, dout), lambda i: (i, 0)),
            pl.BlockSpec((8, dout), lambda i: (0, 0)),
        ],
        out_shape=[
            jax.ShapeDtypeStruct((_PROWS, dout), jnp.float32),
            jax.ShapeDtypeStruct((8, dout), jnp.float32),
        ],
        compiler_params=_SEQ,
    )(hp, aggp, aggp, wbig)


def _pass2(yp, sc, sh, wz, lbz, batrep, zp, dout, gelu_on):
    in_specs = [
        pl.BlockSpec((_RP, dout), lambda i: (i, 0)),
        pl.BlockSpec((1, dout), lambda i: (0, 0)),
        pl.BlockSpec((1, dout), lambda i: (0, 0)),
        pl.BlockSpec((dout, 32), lambda i: (0, 0)),
        pl.BlockSpec((1, 32), lambda i: (0, 0)),
        pl.BlockSpec((_RP, 32), lambda i: (i, 0)),
    ]
    args = [yp, sc, sh, wz, lbz, batrep]
    if zp is not None:
        in_specs.append(pl.BlockSpec((_RP, 32), lambda i: (i, 0)))
        args.append(zp)
    out_specs = [
        pl.BlockSpec((_RP, dout), lambda i: (i, 0)),
        pl.BlockSpec((_RP, 32), lambda i: (i, 0)),
        pl.BlockSpec((_G, 32), lambda i: (0, 0)),
    ]
    out_shape = [
        jax.ShapeDtypeStruct((_PROWS, dout), jnp.float32),
        jax.ShapeDtypeStruct((_PROWS, 32), jnp.float32),
        jax.ShapeDtypeStruct((_G, 32), jnp.float32),
    ]
    return pl.pallas_call(
        functools.partial(_p2_body, gelu_on, zp is not None),
        grid=(_PGRID,),
        in_specs=in_specs,
        out_specs=out_specs,
        out_shape=out_shape,
        compiler_params=_SEQ,
    )(*args)


def _bn_coeffs(st, g, b, dsub, folds=8):
    s = st[0].reshape(folds, dsub).sum(0)
    sq = st[1].reshape(folds, dsub).sum(0)
    m = s * (1.0 / _N)
    v = sq * (1.0 / _N) - m * m
    scale = g * lax.rsqrt(v + 1e-5)
    shift = b - m * scale
    return jnp.tile(scale, 8).reshape(1, 8 * dsub), jnp.tile(shift, 8).reshape(1, 8 * dsub)


def kernel(x, edge_index, batch, W0, b0, bn0_g, bn0_b, lin0_W, lin0_b,
           c1_W, c1_b, bn1_g, bn1_b, lin1_W, lin1_b,
           c2_W, c2_b, bn2_g, bn2_b, lin2_W, lin2_b):
    edges3 = edge_index.reshape(2, _EROWS, 128)
    batrep = jnp.repeat(batch, 4).reshape(_PROWS, 32)
    eye8 = jnp.eye(8, dtype=jnp.float32)

    y0, st0 = _pass1_first(x, W0)
    sc0, sh0 = _bn_coeffs(st0, bn0_g, bn0_b, 16)
    h0, z0, seg0 = _pass2(y0, sc0, sh0, jnp.kron(eye8, lin0_W),
                          jnp.tile(lin0_b, 8).reshape(1, 32), batrep,
                          None, 128, True)

    agg1 = _sc_agg_call(h0.reshape(_N, 16), edges3).reshape(2, _PROWS, 128)
    y1, st1 = _pass1_gin(h0, agg1, jnp.kron(eye8, c1_W), 128)
    sc1, sh1 = _bn_coeffs(st1, bn1_g, bn1_b, 16)
    h1, z1, seg1 = _pass2(y1, sc1, sh1, jnp.kron(eye8, lin1_W),
                          jnp.tile(lin1_b, 8).reshape(1, 32), batrep,
                          z0, 128, False)

    agg2 = _sc_agg_call(h1.reshape(_N, 16), edges3).reshape(2, _PROWS, 128)
    y2, st2 = _pass1_gin(h1, agg2, jnp.kron(eye8, c2_W), 64)
    sc2, sh2 = _bn_coeffs(st2, bn2_g, bn2_b, 8)
    h2, z2, seg2 = _pass2(y2, sc2, sh2, jnp.kron(eye8, lin2_W),
                          jnp.tile(lin2_b, 8).reshape(1, 32), batrep,
                          z1, 64, False)

    segf = lambda s: s.reshape(_G, 8, _T).max(1)
    out = segf(seg0) + segf(seg1) + segf(seg2)
    return (out, z2.reshape(_N, _T), h2.reshape(_N, 8))


# speedup vs baseline: 46.6806x; 1.0016x over previous
import functools

import jax
import jax.numpy as jnp
from jax import lax
from jax.experimental import pallas as pl
from jax.experimental.pallas import tpu as pltpu
from jax.experimental.pallas import tpu_sc as plsc

_N = 100000
_F = 128
_E = 3200000
_G = 8
_T = 4

_NC = 2
_NS = 16
_NW = _NC * _NS
_KCH = 4
_EROWS = _E // 128
_RPT = _EROWS // _NW
_RREM = _EROWS % _NW
_NFULL = _RPT // _KCH
_PER_T = 6250
_N_ACC = _NS * _PER_T
_ZROWS = 250
_CHUNKS = [(o, _ZROWS) for o in range(0, _PER_T, _ZROWS)]

_R = 2000
_INV_SQRT2 = 0.7071067811865476


def _gelu(v):
    return 0.5 * v * (1.0 + lax.erf(v * _INV_SQRT2))


def _sc_agg_call(h, edges3):
    mesh = plsc.VectorSubcoreMesh(core_axis_name="c", subcore_axis_name="s")

    @functools.partial(
        pl.kernel,
        out_type=jax.ShapeDtypeStruct((_NC, _N_ACC, 16), jnp.float32),
        mesh=mesh,
        scratch_types=[
            pltpu.VMEM((_KCH, 128), jnp.int32),
            pltpu.VMEM((_KCH, 128), jnp.int32),
            pltpu.VMEM((_KCH, 128), jnp.int32),
            pltpu.VMEM((_KCH, 128), jnp.int32),
            pltpu.VMEM((_KCH, 128, 16), jnp.float32),
            pltpu.VMEM((_KCH, 128, 16), jnp.float32),
            pltpu.VMEM((_ZROWS, 16), jnp.float32),
            pltpu.VMEM_SHARED((_N_ACC, 16), jnp.float32),
            pltpu.SemaphoreType.DMA,
            pltpu.SemaphoreType.DMA,
            pltpu.SemaphoreType.DMA,
            pltpu.SemaphoreType.DMA,
            pltpu.SemaphoreType.DMA,
            pltpu.SemaphoreType.DMA,
        ],
        compiler_params=pltpu.CompilerParams(use_tc_tiling_on_sc=False),
    )
    def body(h_hbm, e_hbm, out_hbm, src0, dst0, src1, dst1, rows0, rows1,
             zbuf, acc, sg0, sg1, ss0, ss1, si0, si1):
        cid = lax.axis_index("c")
        sid = lax.axis_index("s")
        wid = sid * _NC + cid

        def zrow(i, carry):
            zbuf[i, :] = jnp.zeros((16,), jnp.float32)
            return carry

        lax.fori_loop(0, _ZROWS, zrow, 0)
        for off, sz in _CHUNKS:
            pltpu.sync_copy(zbuf.at[pl.ds(0, sz)],
                            acc.at[pl.ds(sid * _PER_T + off, sz)])
        plsc.subcore_barrier()

        start = wid * _RPT + jnp.minimum(wid, _RREM)
        count = _RPT + (wid < _RREM).astype(jnp.int32)
        bufs = ((src0, dst0, rows0, sg0, ss0, si0),
                (src1, dst1, rows1, sg1, ss1, si1))

        def load_idx(b, row0):
            pltpu.sync_copy(e_hbm.at[0, pl.ds(row0, _KCH)], bufs[b][0])
            pltpu.sync_copy(e_hbm.at[1, pl.ds(row0, _KCH)], bufs[b][1])

        def fire_g(b):
            for j in range(_KCH):
                pltpu.async_copy(h_hbm.at[bufs[b][0].at[j]],
                                 bufs[b][2].at[j], bufs[b][3])

        def wait_g(b):
            for j in range(_KCH):
                pltpu.make_async_copy(h_hbm.at[bufs[b][0].at[j]],
                                      bufs[b][2].at[j], bufs[b][3]).wait()

        def fire_s(b):
            for j in range(_KCH):
                pltpu.async_copy(bufs[b][2].at[j], acc.at[bufs[b][1].at[j]],
                                 bufs[b][4], add=True)

        def wait_s(b):
            for j in range(_KCH):
                pltpu.make_async_copy(bufs[b][2].at[j],
                                      acc.at[bufs[b][1].at[j]],
                                      bufs[b][4]).wait()

        load_idx(0, start)
        fire_g(0)

        def pair(it2, carry):
            it = 2 * it2
            load_idx(1, start + (it + 1) * _KCH)

            @pl.when(it2 > 0)
            def _():
                wait_s(1)

            fire_g(1)
            wait_g(0)
            fire_s(0)
            load_idx(0, start + (it + 2) * _KCH)
            wait_s(0)
            fire_g(0)
            wait_g(1)
            fire_s(1)
            return carry

        lax.fori_loop(0, (_NFULL - 1) // 2, pair, 0)
        wait_g(0)
        fire_s(0)
        wait_s(1)
        wait_s(0)

        def tail(it, carry):
            row = start + _NFULL * _KCH + it
            pltpu.sync_copy(e_hbm.at[0, pl.ds(row, 1)], src0.at[pl.ds(0, 1)])
            pltpu.sync_copy(e_hbm.at[1, pl.ds(row, 1)], dst0.at[pl.ds(0, 1)])
            pltpu.async_copy(h_hbm.at[src0.at[0]], rows0.at[0], sg0).wait()
            pltpu.sync_copy(rows0.at[0], acc.at[dst0.at[0]], add=True)
            return carry

        lax.fori_loop(0, count - _NFULL * _KCH, tail, 0)
        plsc.subcore_barrier()
        for off, sz in _CHUNKS:
            sl = pl.ds(sid * _PER_T + off, sz)
            pltpu.sync_copy(acc.at[sl], out_hbm.at[cid].at[sl])

    return body(h, edges3)


_PROWS = _N // 8
_RP = 1024
_PGRID = -(-_PROWS // _RP)


def _stats_update(i, y, st_ref):
    s = jnp.sum(y, axis=0, keepdims=True)
    sq = jnp.sum(y * y, axis=0, keepdims=True)
    upd = jnp.concatenate([s, sq, jnp.zeros((6, y.shape[1]), jnp.float32)], 0)

    @pl.when(i == 0)
    def _():
        st_ref[...] = jnp.zeros_like(st_ref)

    st_ref[...] += upd


def _p1_first_body(x_ref, w_ref, y_ref, st_ref):
    i = pl.program_id(0)
    w = w_ref[...]
    parts = [
        jnp.dot(x_ref[:, j, :], w, preferred_element_type=jnp.float32)
        for j in range(8)
    ]
    y = jnp.concatenate(parts, axis=1)
    rid = lax.broadcasted_iota(jnp.int32, (y.shape[0], 1), 0) + i * _RP
    y = jnp.where(rid < _PROWS, y, 0.0)
    _stats_update(i, y, st_ref)
    y_ref[...] = y


def _p1_gin_body(h_ref, agga_ref, aggb_ref, w_ref, y_ref, st_ref):
    i = pl.program_id(0)
    hin = h_ref[...] + agga_ref[0] + aggb_ref[0]
    y = jnp.dot(hin, w_ref[...], preferred_element_type=jnp.float32)
    rid = lax.broadcasted_iota(jnp.int32, (y.shape[0], 1), 0) + i * _RP
    y = jnp.where(rid < _PROWS, y, 0.0)
    y_ref[...] = y
    _stats_update(i, y, st_ref)


def _p2_body(gelu_on, has_zp, *refs):
    if has_zp:
        (y_ref, sc_ref, sh_ref, wz_ref, lbz_ref, bat_ref, zp_ref,
         h_ref, z_ref, seg_ref) = refs
    else:
        (y_ref, sc_ref, sh_ref, wz_ref, lbz_ref, bat_ref,
         h_ref, z_ref, seg_ref) = refs
        zp_ref = None
    i = pl.program_id(0)
    h = y_ref[...] * sc_ref[...] + sh_ref[...]
    if gelu_on:
        h = _gelu(h)
    h_ref[...] = h
    z = jnp.dot(h, wz_ref[...], preferred_element_type=jnp.float32) + lbz_ref[...]
    if gelu_on:
        z = _gelu(z)
    z_ref[...] = z + zp_ref[...] if zp_ref is not None else z
    bat = bat_ref[...]
    rid = lax.broadcasted_iota(jnp.int32, (z.shape[0], 1), 0) + i * _RP
    valid = rid < _PROWS
    rows = [
        jnp.max(jnp.where((bat == g) & valid, z, -jnp.inf), axis=0, keepdims=True)
        for g in range(_G)
    ]
    cur = jnp.concatenate(rows, 0)

    @pl.when(i == 0)
    def _():
        seg_ref[...] = jnp.full_like(seg_ref, -jnp.inf)

    seg_ref[...] = jnp.maximum(seg_ref[...], cur)


_SEQ = pltpu.CompilerParams(dimension_semantics=("arbitrary",))


def _pass1_first(x, w):
    return pl.pallas_call(
        _p1_first_body,
        grid=(_PGRID,),
        in_specs=[
            pl.BlockSpec((_RP, 8, _F), lambda i: (i, 0, 0)),
            pl.BlockSpec((_F, 16), lambda i: (0, 0)),
        ],
        out_specs=[
            pl.BlockSpec((_RP, 128), lambda i: (i, 0)),
            pl.BlockSpec((8, 128), lambda i: (0, 0)),
        ],
        out_shape=[
            jax.ShapeDtypeStruct((_PROWS, 128), jnp.float32),
            jax.ShapeDtypeStruct((8, 128), jnp.float32),
        ],
        compiler_params=_SEQ,
    )(x.reshape(_PROWS, 8, _F), w)


def _pass1_gin(hp, aggp, wbig, dout):
    return pl.pallas_call(
        _p1_gin_body,
        grid=(_PGRID,),
        in_specs=[
            pl.BlockSpec((_RP, 128), lambda i: (i, 0)),
            pl.BlockSpec((1, _RP, 128), lambda i: (0, i, 0)),
            pl.BlockSpec((1, _RP, 128), lambda i: (1, i, 0)),
            pl.BlockSpec((128, dout), lambda i: (0, 0)),
        ],
        out_specs=[
            pl.BlockSpec((_RP, dout), lambda i: (i, 0)),
            pl.BlockSpec((8, dout), lambda i: (0, 0)),
        ],
        out_shape=[
            jax.ShapeDtypeStruct((_PROWS, dout), jnp.float32),
            jax.ShapeDtypeStruct((8, dout), jnp.float32),
        ],
        compiler_params=_SEQ,
    )(hp, aggp, aggp, wbig)


def _pass2(yp, sc, sh, wz, lbz, batrep, zp, dout, gelu_on):
    in_specs = [
        pl.BlockSpec((_RP, dout), lambda i: (i, 0)),
        pl.BlockSpec((1, dout), lambda i: (0, 0)),
        pl.BlockSpec((1, dout), lambda i: (0, 0)),
        pl.BlockSpec((dout, 32), lambda i: (0, 0)),
        pl.BlockSpec((1, 32), lambda i: (0, 0)),
        pl.BlockSpec((_RP, 32), lambda i: (i, 0)),
    ]
    args = [yp, sc, sh, wz, lbz, batrep]
    if zp is not None:
        in_specs.append(pl.BlockSpec((_RP, 32), lambda i: (i, 0)))
        args.append(zp)
    out_specs = [
        pl.BlockSpec((_RP, dout), lambda i: (i, 0)),
        pl.BlockSpec((_RP, 32), lambda i: (i, 0)),
        pl.BlockSpec((_G, 32), lambda i: (0, 0)),
    ]
    out_shape = [
        jax.ShapeDtypeStruct((_PROWS, dout), jnp.float32),
        jax.ShapeDtypeStruct((_PROWS, 32), jnp.float32),
        jax.ShapeDtypeStruct((_G, 32), jnp.float32),
    ]
    return pl.pallas_call(
        functools.partial(_p2_body, gelu_on, zp is not None),
        grid=(_PGRID,),
        in_specs=in_specs,
        out_specs=out_specs,
        out_shape=out_shape,
        compiler_params=_SEQ,
    )(*args)


def _bn_coeffs(st, g, b, dsub, folds=8):
    s = st[0].reshape(folds, dsub).sum(0)
    sq = st[1].reshape(folds, dsub).sum(0)
    m = s * (1.0 / _N)
    v = sq * (1.0 / _N) - m * m
    scale = g * lax.rsqrt(v + 1e-5)
    shift = b - m * scale
    return jnp.tile(scale, 8).reshape(1, 8 * dsub), jnp.tile(shift, 8).reshape(1, 8 * dsub)


def kernel(x, edge_index, batch, W0, b0, bn0_g, bn0_b, lin0_W, lin0_b,
           c1_W, c1_b, bn1_g, bn1_b, lin1_W, lin1_b,
           c2_W, c2_b, bn2_g, bn2_b, lin2_W, lin2_b):
    edges3 = edge_index.reshape(2, _EROWS, 128)
    batrep = jnp.repeat(batch, 4).reshape(_PROWS, 32)
    eye8 = jnp.eye(8, dtype=jnp.float32)

    y0, st0 = _pass1_first(x, W0)
    sc0, sh0 = _bn_coeffs(st0, bn0_g, bn0_b, 16)
    h0, z0, seg0 = _pass2(y0, sc0, sh0, jnp.kron(eye8, lin0_W),
                          jnp.tile(lin0_b, 8).reshape(1, 32), batrep,
                          None, 128, True)

    agg1 = _sc_agg_call(h0.reshape(_N, 16), edges3).reshape(2, _PROWS, 128)
    y1, st1 = _pass1_gin(h0, agg1, jnp.kron(eye8, c1_W), 128)
    sc1, sh1 = _bn_coeffs(st1, bn1_g, bn1_b, 16)
    h1, z1, seg1 = _pass2(y1, sc1, sh1, jnp.kron(eye8, lin1_W),
                          jnp.tile(lin1_b, 8).reshape(1, 32), batrep,
                          z0, 128, False)

    agg2 = _sc_agg_call(h1.reshape(_N, 16), edges3).reshape(2, _PROWS, 128)
    y2, st2 = _pass1_gin(h1, agg2, jnp.kron(eye8, c2_W), 64)
    sc2, sh2 = _bn_coeffs(st2, bn2_g, bn2_b, 8)
    h2, z2, seg2 = _pass2(y2, sc2, sh2, jnp.kron(eye8, lin2_W),
                          jnp.tile(lin2_b, 8).reshape(1, 32), batrep,
                          z1, 64, False)

    segf = lambda s: s.reshape(_G, 8, _T).max(1)
    out = segf(seg0) + segf(seg1) + segf(seg2)
    return (out, z2.reshape(_N, _T), h2.reshape(_N, 8))
